# baseline (speedup 1.0000x reference)
import functools

import jax
import jax.numpy as jnp
from jax import lax
from jax.experimental import pallas as pl
from jax.experimental.pallas import tpu as pltpu
from jax.experimental.pallas import tpu_sc as plsc

VOCAB = 1000000
LENGTH = 200
DIM = 64
BATCH = 4096

B = BATCH * LENGTH
NC, NS = 2, 16
NW = NC * NS
BPW = B // NW
STEP = 128
CHUNK_STEPS = 5
CHUNK = STEP * CHUNK_STEPS
NCHUNK = BPW // CHUNK
NSTEPS = BPW // STEP

_mesh = plsc.VectorSubcoreMesh(core_axis_name="c", subcore_axis_name="s")


@functools.partial(
    pl.kernel,
    out_type=jax.ShapeDtypeStruct((B, DIM), jnp.float32),
    mesh=_mesh,
    scratch_types=[
        pltpu.VMEM((NSTEPS, STEP), jnp.int32),
        pltpu.VMEM((LENGTH, DIM), jnp.float32),
        pltpu.VMEM((CHUNK, DIM), jnp.float32),
        pltpu.SemaphoreType.DMA,
    ],
    compiler_params=pltpu.CompilerParams(use_tc_tiling_on_sc=False),
)
def _embed(x_hbm, tok_hbm, pos_hbm, out_hbm, idx_v, pos_v, dest_v, sem):
    wid = lax.axis_index("s") * NC + lax.axis_index("c")
    base = wid * BPW

    pltpu.sync_copy(x_hbm.at[pl.ds(wid * NSTEPS, NSTEPS)], idx_v)
    pltpu.sync_copy(pos_hbm, pos_v)

    @pl.loop(0, NCHUNK)
    def _chunk(c):
        copies = []
        for s in range(CHUNK_STEPS):
            copies.append(pltpu.async_copy(
                tok_hbm.at[idx_v.at[c * CHUNK_STEPS + s]],
                dest_v.at[pl.ds(s * STEP, STEP)],
                sem,
            ))
        for cp in copies:
            cp.wait()

        p0 = lax.rem(c * CHUNK, LENGTH)

        @pl.loop(0, CHUNK, init_carry=p0)
        def _add(r, p):
            for d in range(DIM // 16):
                sl = pl.ds(d * 16, 16)
                dest_v[r, sl] = dest_v[r, sl] + pos_v[p, sl]
            p = p + 1
            return jnp.where(p == LENGTH, 0, p)

        pltpu.sync_copy(dest_v, out_hbm.at[pl.ds(base + c * CHUNK, CHUNK)])


def kernel(x, token_table, pos_table):
    xi = x.reshape(B).astype(jnp.int32).reshape(B // STEP, STEP)
    out = _embed(xi, token_table, pos_table)
    return out.reshape(BATCH, LENGTH, DIM)


# speedup vs baseline: 1.3548x; 1.3548x over previous
import functools

import jax
import jax.numpy as jnp
from jax import lax
from jax.experimental import pallas as pl
from jax.experimental.pallas import tpu as pltpu
from jax.experimental.pallas import tpu_sc as plsc

VOCAB = 1000000
LENGTH = 200
DIM = 64
BATCH = 4096

B = BATCH * LENGTH
NC, NS = 2, 16
NW = NC * NS
BPW = B // NW
STEP = 100
SEQ_PER_CHUNK = 2
CHUNK = SEQ_PER_CHUNK * LENGTH
CHUNK_STEPS = CHUNK // STEP
NCHUNK = BPW // CHUNK
NSTEPS = BPW // STEP
NBUF = 2

_mesh = plsc.VectorSubcoreMesh(core_axis_name="c", subcore_axis_name="s")


@functools.partial(
    pl.kernel,
    out_type=jax.ShapeDtypeStruct((B, DIM), jnp.float32),
    mesh=_mesh,
    scratch_types=[
        pltpu.VMEM((NSTEPS, STEP), jnp.int32),
        pltpu.VMEM((LENGTH, DIM), jnp.float32),
        pltpu.VMEM((NBUF, CHUNK, DIM), jnp.float32),
        pltpu.SemaphoreType.DMA((NBUF,)),
    ],
    compiler_params=pltpu.CompilerParams(use_tc_tiling_on_sc=False),
)
def _embed(x_hbm, tok_hbm, pos_hbm, out_hbm, idx_v, pos_v, dest_v, sems):
    wid = lax.axis_index("s") * NC + lax.axis_index("c")
    base = wid * BPW

    pltpu.sync_copy(x_hbm.at[pl.ds(wid * NSTEPS, NSTEPS)], idx_v)
    pltpu.sync_copy(pos_hbm, pos_v)

    def fire(c, b):
        for s in range(CHUNK_STEPS):
            pltpu.async_copy(
                tok_hbm.at[idx_v.at[c * CHUNK_STEPS + s]],
                dest_v.at[b, pl.ds(s * STEP, STEP)],
                sems.at[b],
            )

    def drain(b):
        pltpu.make_async_copy(
            tok_hbm.at[pl.ds(0, CHUNK)], dest_v.at[b], sems.at[b]
        ).wait()

    for b in range(NBUF):
        fire(b, b)

    @pl.loop(0, NCHUNK, step=NBUF)
    def _chunks(c):
        for b in range(NBUF):
            cc = c + b
            drain(b)

            @pl.loop(0, LENGTH)
            def _add(p):
                for q in range(SEQ_PER_CHUNK):
                    row = q * LENGTH + p
                    for d in range(DIM // 16):
                        sl = pl.ds(d * 16, 16)
                        plsc.addupdate(dest_v.at[b, row, sl], pos_v[p, sl])

            pltpu.sync_copy(
                dest_v.at[b], out_hbm.at[pl.ds(base + cc * CHUNK, CHUNK)]
            )

            @pl.when(cc + NBUF < NCHUNK)
            def _refill():
                fire(cc + NBUF, b)


def kernel(x, token_table, pos_table):
    xi = x.reshape(B).astype(jnp.int32).reshape(B // STEP, STEP)
    out = _embed(xi, token_table, pos_table)
    return out.reshape(BATCH, LENGTH, DIM)
